# baseline (speedup 1.0000x reference)
import functools

import jax
import jax.numpy as jnp
from jax import lax
from jax.experimental import pallas as pl
from jax.experimental.pallas import tpu as pltpu
from jax.experimental.pallas import tpu_sc as plsc

BVS = 1000000
MD = BVS - 1
L = 16
NC, NS = 2, 16
NW = NC * NS

BLK = 2048
NBLK = BVS // BLK
TAIL = BVS - NBLK * BLK


def _detile_body(V, D, emt_hbm, out_hbm, slab, stg, slabT, stgT, sems):
    w = lax.axis_index("s") * NC + lax.axis_index("c")

    def do_block(gr, c0, n, buf, sbuf, sem):
        pltpu.async_copy(
            emt_hbm.at[pl.ds(8 * gr, 8), pl.ds(c0, n)], buf, sem
        )
        pltpu.make_async_copy(
            emt_hbm.at[pl.ds(8 * gr, 8), pl.ds(c0, n)], buf, sem
        ).wait()

        def vcopy(i, _):
            for r in range(8):
                sbuf[pl.ds(r * n + i * L, L)] = buf[r, pl.ds(i * L, L)]
            return 0

        lax.fori_loop(0, n // L, vcopy, 0, unroll=4)
        for r in range(8):
            pltpu.async_copy(
                sbuf.at[pl.ds(r * n, n)],
                out_hbm.at[pl.ds((8 * gr + r) * V + c0, n)],
                sem,
            )
        for r in range(8):
            pltpu.make_async_copy(
                sbuf.at[pl.ds(r * n, n)],
                out_hbm.at[pl.ds((8 * gr + r) * V + c0, n)],
                sem,
            ).wait()

    for gr in range(8):
        def blk_body(k, _):
            j = w + NW * k

            @pl.when(j < NBLK)
            def _():
                do_block(gr, j * BLK, BLK, slab, stg, sems.at[0])
            return 0

        lax.fori_loop(0, (NBLK + NW - 1) // NW, blk_body, 0)

        @pl.when(w == gr)
        def _():
            do_block(gr, NBLK * BLK, TAIL, slabT, stgT, sems.at[1])


def _lookup_body(S, N, D, tk_hbm, emt_hbm, sc_hbm, out_hbm,
                 tkbuf, idxv, gbufs, obufs, scv, gsems, osems):
    CHUNK = N // NW
    wid = lax.axis_index("s") * NC + lax.axis_index("c")
    base = wid * CHUNK
    b = base // S
    s0 = base % S

    pltpu.sync_copy(sc_hbm, scv)
    pltpu.sync_copy(tk_hbm.at[pl.ds(base, CHUNK)], tkbuf.at[pl.ds(8, CHUNK)])

    @pl.when(base != 0)
    def _():
        pltpu.sync_copy(tk_hbm.at[pl.ds(base - 8, 8)], tkbuf.at[pl.ds(0, 8)])

    def hash_body(i, _):
        cur = tkbuf[pl.ds(8 + i * L, L)]
        prev = tkbuf[pl.ds(7 + i * L, L)]
        a = jnp.int32(36313) * cur
        bb = jnp.int32(27191) * prev
        r = lax.rem(lax.bitwise_xor(a, bb), jnp.int32(MD))
        r = jnp.where(r < 0, r + jnp.int32(MD), r)
        pos = base + i * L + lax.iota(jnp.int32, L)
        r = jnp.where((pos & (S - 1)) == 0, jnp.int32(MD), r)
        idxv[pl.ds(i * L, L)] = r
        return 0

    lax.fori_loop(0, CHUNK // L, hash_body, 0, unroll=2)

    scale = scv[...]
    obase0 = b * D * S + s0

    def gather_d(d, buf):
        pltpu.async_copy(
            emt_hbm.at[d].at[idxv], gbufs.at[buf], gsems.at[buf]
        )

    for buf in range(2):
        gather_d(buf, buf)

    def step(jo, _):
        for buf in range(2):
            d = 2 * jo + buf
            pltpu.make_async_copy(
                emt_hbm.at[d].at[idxv], gbufs.at[buf], gsems.at[buf]
            ).wait()

            @pl.when(d >= 2)
            def _():
                pltpu.make_async_copy(
                    obufs.at[buf],
                    out_hbm.at[pl.ds(obase0 + (d - 2) * S, CHUNK)],
                    osems.at[buf],
                ).wait()

            def sbody(i, _):
                obufs[buf, pl.ds(i * L, L)] = (
                    gbufs[buf, pl.ds(i * L, L)] * scale
                )
                return 0

            lax.fori_loop(0, CHUNK // L, sbody, 0, unroll=4)

            @pl.when(d + 2 < D)
            def _():
                gather_d(d + 2, buf)

            pltpu.async_copy(
                obufs.at[buf],
                out_hbm.at[pl.ds(obase0 + d * S, CHUNK)],
                osems.at[buf],
            )
        return 0

    lax.fori_loop(0, D // 2, step, 0)

    for buf in range(2):
        pltpu.make_async_copy(
            obufs.at[buf],
            out_hbm.at[pl.ds(obase0, CHUNK)],
            osems.at[buf],
        ).wait()


def kernel(tk, em_weight, sc):
    B, S = tk.shape
    V, D = em_weight.shape
    N = B * S
    CHUNK = N // NW

    tk_flat = tk.reshape(N).astype(jnp.int32)
    emt = em_weight.T
    sc_vec = jnp.broadcast_to(sc.astype(jnp.float32), (L,))

    mesh = plsc.VectorSubcoreMesh(core_axis_name="c", subcore_axis_name="s")

    emt_lin = pl.kernel(
        functools.partial(_detile_body, V, D),
        mesh=mesh,
        compiler_params=pltpu.CompilerParams(use_tc_tiling_on_sc=True),
        out_type=jax.ShapeDtypeStruct((D * V,), jnp.float32),
        scratch_types=[
            pltpu.VMEM((8, BLK), jnp.float32),
            pltpu.VMEM((8 * BLK,), jnp.float32),
            pltpu.VMEM((8, TAIL), jnp.float32),
            pltpu.VMEM((8 * TAIL,), jnp.float32),
            pltpu.SemaphoreType.DMA((2,)),
        ],
    )(emt)

    body = functools.partial(_lookup_body, S, N, D)
    out = pl.kernel(
        body,
        mesh=mesh,
        compiler_params=pltpu.CompilerParams(use_tc_tiling_on_sc=False),
        out_type=jax.ShapeDtypeStruct((B * D * S,), jnp.float32),
        scratch_types=[
            pltpu.VMEM((CHUNK + 8,), jnp.int32),
            pltpu.VMEM((CHUNK,), jnp.int32),
            pltpu.VMEM((2, CHUNK), jnp.float32),
            pltpu.VMEM((2, CHUNK), jnp.float32),
            pltpu.VMEM((L,), jnp.float32),
            pltpu.SemaphoreType.DMA((2,)),
            pltpu.SemaphoreType.DMA((2,)),
        ],
    )(tk_flat, emt_lin.reshape(D, V), sc_vec)
    return out.reshape(B, D, S).transpose(0, 2, 1)


# speedup vs baseline: 2.5988x; 2.5988x over previous
import functools

import jax
import jax.numpy as jnp
from jax import lax
from jax.experimental import pallas as pl
from jax.experimental.pallas import tpu as pltpu
from jax.experimental.pallas import tpu_sc as plsc

BVS = 1000000
MD = BVS - 1
L = 16
NC, NS = 2, 16
NW = NC * NS

CW = 131072
NCH = (BVS + CW - 1) // CW


def _tc_detile_body(emt_ref, out_ref):
    def body(ct, _):
        out_ref[0, 0, ct] = emt_ref[:, pl.ds(pl.multiple_of(ct * 128, 128), 128)]
        return 0

    lax.fori_loop(0, CW // 128, body, 0, unroll=4)


def _lookup_body(S, N, D, tk_hbm, emt_hbm, sc_hbm, out_hbm,
                 tkbuf, idxv, gbufs, obufs, scv, gsems, osems):
    CHUNK = N // NW
    wid = lax.axis_index("s") * NC + lax.axis_index("c")
    base = wid * CHUNK
    b = base // S
    s0 = base % S

    pltpu.sync_copy(sc_hbm, scv)
    pltpu.sync_copy(tk_hbm.at[pl.ds(base, CHUNK)], tkbuf.at[pl.ds(8, CHUNK)])

    @pl.when(base != 0)
    def _():
        pltpu.sync_copy(tk_hbm.at[pl.ds(base - 8, 8)], tkbuf.at[pl.ds(0, 8)])

    def hash_body(i, _):
        cur = tkbuf[pl.ds(8 + i * L, L)]
        prev = tkbuf[pl.ds(7 + i * L, L)]
        a = jnp.int32(36313) * cur
        bb = jnp.int32(27191) * prev
        r = lax.rem(lax.bitwise_xor(a, bb), jnp.int32(MD))
        r = jnp.where(r < 0, r + jnp.int32(MD), r)
        pos = base + i * L + lax.iota(jnp.int32, L)
        v = jnp.where((pos & (S - 1)) == 0, jnp.int32(MD), r)
        e = (
            lax.shift_left(lax.shift_right_logical(v, 17), 20)
            + lax.shift_left(
                lax.bitwise_and(lax.shift_right_logical(v, 7), jnp.int32(1023)),
                10,
            )
            + (v & 127)
        )
        for rr in range(8):
            idxv[rr, pl.ds(i * L, L)] = e + rr * 128
        return 0

    lax.fori_loop(0, CHUNK // L, hash_body, 0, unroll=2)

    scale = scv[...]
    obase0 = b * D * S + s0

    def gather_d(d, buf):
        pltpu.async_copy(
            emt_hbm.at[d // 8].at[idxv.at[d % 8]],
            gbufs.at[buf], gsems.at[buf],
        )

    for buf in range(2):
        gather_d(buf, buf)

    def step(jo, _):
        for buf in range(2):
            d = 2 * jo + buf
            pltpu.make_async_copy(
                emt_hbm.at[d // 8].at[idxv.at[d % 8]],
                gbufs.at[buf], gsems.at[buf],
            ).wait()

            @pl.when(d >= 2)
            def _():
                pltpu.make_async_copy(
                    obufs.at[buf],
                    out_hbm.at[pl.ds(obase0 + (d - 2) * S, CHUNK)],
                    osems.at[buf],
                ).wait()

            def sbody(i, _):
                obufs[buf, pl.ds(i * L, L)] = (
                    gbufs[buf, pl.ds(i * L, L)] * scale
                )
                return 0

            lax.fori_loop(0, CHUNK // L, sbody, 0, unroll=4)

            @pl.when(d + 2 < D)
            def _():
                gather_d(d + 2, buf)

            pltpu.async_copy(
                obufs.at[buf],
                out_hbm.at[pl.ds(obase0 + d * S, CHUNK)],
                osems.at[buf],
            )
        return 0

    lax.fori_loop(0, D // 2, step, 0)

    for buf in range(2):
        pltpu.make_async_copy(
            obufs.at[buf],
            out_hbm.at[pl.ds(obase0, CHUNK)],
            osems.at[buf],
        ).wait()


def kernel(tk, em_weight, sc):
    B, S = tk.shape
    V, D = em_weight.shape
    N = B * S
    CHUNK = N // NW

    tk_flat = tk.reshape(N).astype(jnp.int32)
    emt = em_weight.T
    sc_vec = jnp.broadcast_to(sc.astype(jnp.float32), (L,))

    emt_lin = pl.pallas_call(
        _tc_detile_body,
        grid=(D // 8, NCH),
        in_specs=[pl.BlockSpec((8, CW), lambda d8, ch: (d8, ch))],
        out_specs=pl.BlockSpec(
            (1, 1, CW // 128, 8, 128), lambda d8, ch: (d8, ch, 0, 0, 0)
        ),
        out_shape=jax.ShapeDtypeStruct(
            (D // 8, NCH, CW // 128, 8, 128), jnp.float32
        ),
        compiler_params=pltpu.CompilerParams(
            dimension_semantics=("arbitrary", "arbitrary"),
        ),
    )(emt)

    mesh = plsc.VectorSubcoreMesh(core_axis_name="c", subcore_axis_name="s")
    body = functools.partial(_lookup_body, S, N, D)
    out = pl.kernel(
        body,
        mesh=mesh,
        compiler_params=pltpu.CompilerParams(use_tc_tiling_on_sc=False),
        out_type=jax.ShapeDtypeStruct((B * D * S,), jnp.float32),
        scratch_types=[
            pltpu.VMEM((CHUNK + 8,), jnp.int32),
            pltpu.VMEM((8, CHUNK), jnp.int32),
            pltpu.VMEM((2, CHUNK), jnp.float32),
            pltpu.VMEM((2, CHUNK), jnp.float32),
            pltpu.VMEM((L,), jnp.float32),
            pltpu.SemaphoreType.DMA((2,)),
            pltpu.SemaphoreType.DMA((2,)),
        ],
    )(tk_flat, emt_lin.reshape(D // 8, NCH * 8 * CW), sc_vec)
    return out.reshape(B, D, S).transpose(0, 2, 1)


# speedup vs baseline: 2.9202x; 1.1237x over previous
import functools

import jax
import jax.numpy as jnp
from jax import lax
from jax.experimental import pallas as pl
from jax.experimental.pallas import tpu as pltpu
from jax.experimental.pallas import tpu_sc as plsc

BVS = 1000000
MD = BVS - 1
L = 16
NC, NS = 2, 16
NW = NC * NS

CW = 131072
NCH = (BVS + CW - 1) // CW


def _tc_detile_body(emt_ref, out_ref):
    x = emt_ref[...]
    out_ref[0, 0] = jnp.swapaxes(x.reshape(8, CW // 128, 128), 0, 1)


def _lookup_body(S, N, D, tk_hbm, emt_hbm, sc_hbm, out_hbm,
                 tkbuf, idxv, gbufs, obufs, scv, gsems, osems):
    CHUNK = N // NW
    wid = lax.axis_index("s") * NC + lax.axis_index("c")
    base = wid * CHUNK
    b = base // S
    s0 = base % S

    pltpu.sync_copy(sc_hbm, scv)
    pltpu.sync_copy(tk_hbm.at[pl.ds(base, CHUNK)], tkbuf.at[pl.ds(8, CHUNK)])

    @pl.when(base != 0)
    def _():
        pltpu.sync_copy(tk_hbm.at[pl.ds(base - 8, 8)], tkbuf.at[pl.ds(0, 8)])

    def hash_body(i, _):
        cur = tkbuf[pl.ds(8 + i * L, L)]
        prev = tkbuf[pl.ds(7 + i * L, L)]
        a = jnp.int32(36313) * cur
        bb = jnp.int32(27191) * prev
        r = lax.rem(lax.bitwise_xor(a, bb), jnp.int32(MD))
        r = jnp.where(r < 0, r + jnp.int32(MD), r)
        pos = base + i * L + lax.iota(jnp.int32, L)
        v = jnp.where((pos & (S - 1)) == 0, jnp.int32(MD), r)
        e = (
            lax.shift_left(lax.shift_right_logical(v, 17), 20)
            + lax.shift_left(
                lax.bitwise_and(lax.shift_right_logical(v, 7), jnp.int32(1023)),
                10,
            )
            + (v & 127)
        )
        for rr in range(8):
            idxv[rr, pl.ds(i * L, L)] = e + rr * 128
        return 0

    lax.fori_loop(0, CHUNK // L, hash_body, 0, unroll=2)

    scale = scv[...]
    obase0 = b * D * S + s0

    def gather_d(d, buf):
        pltpu.async_copy(
            emt_hbm.at[d // 8].at[idxv.at[d % 8]],
            gbufs.at[buf], gsems.at[buf],
        )

    for buf in range(2):
        gather_d(buf, buf)

    def step(jo, _):
        for buf in range(2):
            d = 2 * jo + buf
            pltpu.make_async_copy(
                emt_hbm.at[d // 8].at[idxv.at[d % 8]],
                gbufs.at[buf], gsems.at[buf],
            ).wait()

            @pl.when(d >= 2)
            def _():
                pltpu.make_async_copy(
                    obufs.at[buf],
                    out_hbm.at[pl.ds(obase0 + (d - 2) * S, CHUNK)],
                    osems.at[buf],
                ).wait()

            def sbody(i, _):
                obufs[buf, pl.ds(i * L, L)] = (
                    gbufs[buf, pl.ds(i * L, L)] * scale
                )
                return 0

            lax.fori_loop(0, CHUNK // L, sbody, 0, unroll=4)

            @pl.when(d + 2 < D)
            def _():
                gather_d(d + 2, buf)

            pltpu.async_copy(
                obufs.at[buf],
                out_hbm.at[pl.ds(obase0 + d * S, CHUNK)],
                osems.at[buf],
            )
        return 0

    lax.fori_loop(0, D // 2, step, 0)

    for buf in range(2):
        pltpu.make_async_copy(
            obufs.at[buf],
            out_hbm.at[pl.ds(obase0, CHUNK)],
            osems.at[buf],
        ).wait()


def kernel(tk, em_weight, sc):
    B, S = tk.shape
    V, D = em_weight.shape
    N = B * S
    CHUNK = N // NW

    tk_flat = tk.reshape(N).astype(jnp.int32)
    emt = em_weight.T
    sc_vec = jnp.broadcast_to(sc.astype(jnp.float32), (L,))

    emt_lin = pl.pallas_call(
        _tc_detile_body,
        grid=(D // 8, NCH),
        in_specs=[pl.BlockSpec((8, CW), lambda d8, ch: (d8, ch))],
        out_specs=pl.BlockSpec(
            (1, 1, CW // 128, 8, 128), lambda d8, ch: (d8, ch, 0, 0, 0)
        ),
        out_shape=jax.ShapeDtypeStruct(
            (D // 8, NCH, CW // 128, 8, 128), jnp.float32
        ),
        compiler_params=pltpu.CompilerParams(
            dimension_semantics=("arbitrary", "arbitrary"),
        ),
    )(emt)

    mesh = plsc.VectorSubcoreMesh(core_axis_name="c", subcore_axis_name="s")
    body = functools.partial(_lookup_body, S, N, D)
    out = pl.kernel(
        body,
        mesh=mesh,
        compiler_params=pltpu.CompilerParams(use_tc_tiling_on_sc=False),
        out_type=jax.ShapeDtypeStruct((B * D * S,), jnp.float32),
        scratch_types=[
            pltpu.VMEM((CHUNK + 8,), jnp.int32),
            pltpu.VMEM((8, CHUNK), jnp.int32),
            pltpu.VMEM((2, CHUNK), jnp.float32),
            pltpu.VMEM((2, CHUNK), jnp.float32),
            pltpu.VMEM((L,), jnp.float32),
            pltpu.SemaphoreType.DMA((2,)),
            pltpu.SemaphoreType.DMA((2,)),
        ],
    )(tk_flat, emt_lin.reshape(D // 8, NCH * 8 * CW), sc_vec)
    return out.reshape(B, D, S).transpose(0, 2, 1)


# speedup vs baseline: 3.1100x; 1.0650x over previous
import functools

import jax
import jax.numpy as jnp
from jax import lax
from jax.experimental import pallas as pl
from jax.experimental.pallas import tpu as pltpu
from jax.experimental.pallas import tpu_sc as plsc

BVS = 1000000
MD = BVS - 1
L = 16
NC, NS = 2, 16
NW = NC * NS

CW = 262144
CWLOG = 18
NB = 4
NCH = (BVS + CW - 1) // CW


def _tc_detile_body(emt_ref, out_ref):
    x = emt_ref[...]
    out_ref[0, 0] = jnp.swapaxes(x.reshape(8, CW // 128, 128), 0, 1)


def _lookup_body(S, N, D, tk_hbm, emt_hbm, sc_hbm, out_hbm,
                 tkbuf, idxv, gbufs, obufs, scv, gsems, osems):
    CHUNK = N // NW
    wid = lax.axis_index("s") * NC + lax.axis_index("c")
    base = wid * CHUNK
    b = base // S
    s0 = base % S

    pltpu.sync_copy(sc_hbm, scv)
    pltpu.sync_copy(tk_hbm.at[pl.ds(base, CHUNK)], tkbuf.at[pl.ds(8, CHUNK)])

    @pl.when(base != 0)
    def _():
        pltpu.sync_copy(tk_hbm.at[pl.ds(base - 8, 8)], tkbuf.at[pl.ds(0, 8)])

    def hash_body(i, _):
        cur = tkbuf[pl.ds(8 + i * L, L)]
        prev = tkbuf[pl.ds(7 + i * L, L)]
        a = jnp.int32(36313) * cur
        bb = jnp.int32(27191) * prev
        r = lax.rem(lax.bitwise_xor(a, bb), jnp.int32(MD))
        r = jnp.where(r < 0, r + jnp.int32(MD), r)
        pos = base + i * L + lax.iota(jnp.int32, L)
        v = jnp.where((pos & (S - 1)) == 0, jnp.int32(MD), r)
        e = (
            lax.shift_left(lax.shift_right_logical(v, CWLOG), CWLOG + 3)
            + lax.shift_left(
                lax.bitwise_and(
                    lax.shift_right_logical(v, 7), jnp.int32(CW // 128 - 1)
                ),
                10,
            )
            + (v & 127)
        )
        for rr in range(8):
            idxv[rr, pl.ds(i * L, L)] = e + rr * 128
        return 0

    lax.fori_loop(0, CHUNK // L, hash_body, 0, unroll=2)

    scale = scv[...]
    obase0 = b * D * S + s0

    def gather_d(d, buf):
        pltpu.async_copy(
            emt_hbm.at[d // 8].at[idxv.at[d % 8]],
            gbufs.at[buf], gsems.at[buf],
        )

    for buf in range(NB):
        gather_d(buf, buf)

    def step(jo, _):
        for buf in range(NB):
            d = NB * jo + buf
            pltpu.make_async_copy(
                emt_hbm.at[d // 8].at[idxv.at[d % 8]],
                gbufs.at[buf], gsems.at[buf],
            ).wait()

            @pl.when(d >= NB)
            def _():
                pltpu.make_async_copy(
                    obufs.at[buf],
                    out_hbm.at[pl.ds(obase0 + (d - NB) * S, CHUNK)],
                    osems.at[buf],
                ).wait()

            def sbody(i, _):
                obufs[buf, pl.ds(i * L, L)] = (
                    gbufs[buf, pl.ds(i * L, L)] * scale
                )
                return 0

            lax.fori_loop(0, CHUNK // L, sbody, 0, unroll=4)

            @pl.when(d + NB < D)
            def _():
                gather_d(d + NB, buf)

            pltpu.async_copy(
                obufs.at[buf],
                out_hbm.at[pl.ds(obase0 + d * S, CHUNK)],
                osems.at[buf],
            )
        return 0

    lax.fori_loop(0, D // NB, step, 0)

    for buf in range(NB):
        pltpu.make_async_copy(
            obufs.at[buf],
            out_hbm.at[pl.ds(obase0, CHUNK)],
            osems.at[buf],
        ).wait()


def kernel(tk, em_weight, sc):
    B, S = tk.shape
    V, D = em_weight.shape
    N = B * S
    CHUNK = N // NW

    tk_flat = tk.reshape(N).astype(jnp.int32)
    emt = em_weight.T
    sc_vec = jnp.broadcast_to(sc.astype(jnp.float32), (L,))

    emt_lin = pl.pallas_call(
        _tc_detile_body,
        grid=(D // 8, NCH),
        in_specs=[pl.BlockSpec((8, CW), lambda d8, ch: (d8, ch))],
        out_specs=pl.BlockSpec(
            (1, 1, CW // 128, 8, 128), lambda d8, ch: (d8, ch, 0, 0, 0)
        ),
        out_shape=jax.ShapeDtypeStruct(
            (D // 8, NCH, CW // 128, 8, 128), jnp.float32
        ),
        compiler_params=pltpu.CompilerParams(
            dimension_semantics=("arbitrary", "arbitrary"),
        ),
    )(emt)

    mesh = plsc.VectorSubcoreMesh(core_axis_name="c", subcore_axis_name="s")
    body = functools.partial(_lookup_body, S, N, D)
    out = pl.kernel(
        body,
        mesh=mesh,
        compiler_params=pltpu.CompilerParams(use_tc_tiling_on_sc=False),
        out_type=jax.ShapeDtypeStruct((B * D * S,), jnp.float32),
        scratch_types=[
            pltpu.VMEM((CHUNK + 8,), jnp.int32),
            pltpu.VMEM((8, CHUNK), jnp.int32),
            pltpu.VMEM((NB, CHUNK), jnp.float32),
            pltpu.VMEM((NB, CHUNK), jnp.float32),
            pltpu.VMEM((L,), jnp.float32),
            pltpu.SemaphoreType.DMA((NB,)),
            pltpu.SemaphoreType.DMA((NB,)),
        ],
    )(tk_flat, emt_lin.reshape(D // 8, NCH * 8 * CW), sc_vec)
    return out.reshape(B, D, S).transpose(0, 2, 1)


# speedup vs baseline: 3.1364x; 1.0085x over previous
import functools

import jax
import jax.numpy as jnp
from jax import lax
from jax.experimental import pallas as pl
from jax.experimental.pallas import tpu as pltpu
from jax.experimental.pallas import tpu_sc as plsc

BVS = 1000000
MD = BVS - 1
L = 16
NC, NS = 2, 16
NW = NC * NS

CW = 262144
CWLOG = 18
NB = 4
NCH = (BVS + CW - 1) // CW


def _tc_detile_body(emt_ref, out_ref):
    x = emt_ref[...]
    out_ref[0, 0] = jnp.swapaxes(x.reshape(8, CW // 128, 128), 0, 1)


def _lookup_body(S, N, D, tk_hbm, emt_hbm, sc_hbm, out_hbm,
                 tkbuf, idxv, gbufs, obufs, scv, gsems, osems):
    CHUNK = N // NW
    wid = lax.axis_index("s") * NC + lax.axis_index("c")
    base = wid * CHUNK
    b = base // S
    s0 = base % S

    pltpu.sync_copy(sc_hbm, scv)
    pltpu.sync_copy(tk_hbm.at[pl.ds(base, CHUNK)], tkbuf.at[pl.ds(8, CHUNK)])

    @pl.when(base != 0)
    def _():
        pltpu.sync_copy(tk_hbm.at[pl.ds(base - 8, 8)], tkbuf.at[pl.ds(0, 8)])

    def hash_body(i, _):
        cur = tkbuf[pl.ds(8 + i * L, L)]
        prev = tkbuf[pl.ds(7 + i * L, L)]
        a = jnp.int32(36313) * cur
        bb = jnp.int32(27191) * prev
        r = lax.rem(lax.bitwise_xor(a, bb), jnp.int32(MD))
        r = jnp.where(r < 0, r + jnp.int32(MD), r)
        pos = base + i * L + lax.iota(jnp.int32, L)
        v = jnp.where((pos & (S - 1)) == 0, jnp.int32(MD), r)
        e = (
            lax.shift_left(lax.shift_right_logical(v, CWLOG), CWLOG + 3)
            + lax.shift_left(
                lax.bitwise_and(
                    lax.shift_right_logical(v, 7), jnp.int32(CW // 128 - 1)
                ),
                10,
            )
            + (v & 127)
        )
        for rr in range(8):
            idxv[rr, pl.ds(i * L, L)] = e + rr * 128
        return 0

    lax.fori_loop(0, CHUNK // L, hash_body, 0, unroll=2)

    scale = scv[...]
    obase0 = b * D * S + s0

    def gather_d(d, buf):
        pltpu.async_copy(
            emt_hbm.at[d // 8].at[idxv.at[d % 8]],
            gbufs.at[buf], gsems.at[buf],
        )

    for buf in range(NB):
        gather_d(buf, buf)

    def step(jo, _):
        for buf in range(NB):
            d = NB * jo + buf
            pltpu.make_async_copy(
                emt_hbm.at[d // 8].at[idxv.at[d % 8]],
                gbufs.at[buf], gsems.at[buf],
            ).wait()

            @pl.when(d >= NB)
            def _():
                pltpu.make_async_copy(
                    obufs.at[buf],
                    out_hbm.at[pl.ds(obase0 + (d - NB) * S, CHUNK)],
                    osems.at[buf],
                ).wait()

            def sbody(i, _):
                obufs[buf, pl.ds(i * L, L)] = (
                    gbufs[buf, pl.ds(i * L, L)] * scale
                )
                return 0

            lax.fori_loop(0, CHUNK // L, sbody, 0, unroll=4)

            @pl.when(d + NB < D)
            def _():
                gather_d(d + NB, buf)

            pltpu.async_copy(
                obufs.at[buf],
                out_hbm.at[pl.ds(obase0 + d * S, CHUNK)],
                osems.at[buf],
            )
        return 0

    lax.fori_loop(0, D // NB, step, 0)

    for buf in range(NB):
        pltpu.make_async_copy(
            obufs.at[buf],
            out_hbm.at[pl.ds(obase0, CHUNK)],
            osems.at[buf],
        ).wait()


def kernel(tk, em_weight, sc):
    B, S = tk.shape
    V, D = em_weight.shape
    N = B * S
    CHUNK = N // NW

    tk_flat = tk.reshape(N).astype(jnp.int32)
    emt = em_weight.T
    sc_vec = jnp.broadcast_to(sc.astype(jnp.float32), (L,))

    mesh = plsc.VectorSubcoreMesh(core_axis_name="c", subcore_axis_name="s")

    def tc_half(off):
        return pl.pallas_call(
            _tc_detile_body,
            grid=(D // 16, NCH),
            in_specs=[
                pl.BlockSpec((8, CW), lambda d8, ch: (d8 + off, ch))
            ],
            out_specs=pl.BlockSpec(
                (1, 1, CW // 128, 8, 128), lambda d8, ch: (d8, ch, 0, 0, 0)
            ),
            out_shape=jax.ShapeDtypeStruct(
                (D // 16, NCH, CW // 128, 8, 128), jnp.float32
            ),
            compiler_params=pltpu.CompilerParams(
                dimension_semantics=("arbitrary", "arbitrary"),
            ),
        )(emt)

    def sc_half(emt_lin):
        body = functools.partial(_lookup_body, S, N, D // 2)
        return pl.kernel(
            body,
            mesh=mesh,
            compiler_params=pltpu.CompilerParams(use_tc_tiling_on_sc=False),
            out_type=jax.ShapeDtypeStruct((B * (D // 2) * S,), jnp.float32),
            scratch_types=[
                pltpu.VMEM((CHUNK + 8,), jnp.int32),
                pltpu.VMEM((8, CHUNK), jnp.int32),
                pltpu.VMEM((NB, CHUNK), jnp.float32),
                pltpu.VMEM((NB, CHUNK), jnp.float32),
                pltpu.VMEM((L,), jnp.float32),
                pltpu.SemaphoreType.DMA((NB,)),
                pltpu.SemaphoreType.DMA((NB,)),
            ],
        )(tk_flat, emt_lin.reshape(D // 16, NCH * 8 * CW), sc_vec)

    halves = [sc_half(tc_half(h * (D // 16))) for h in range(2)]
    out = jnp.concatenate(
        [o.reshape(B, D // 2, S) for o in halves], axis=1
    )
    return out.transpose(0, 2, 1)


# speedup vs baseline: 3.1372x; 1.0003x over previous
import functools

import jax
import jax.numpy as jnp
from jax import lax
from jax.experimental import pallas as pl
from jax.experimental.pallas import tpu as pltpu
from jax.experimental.pallas import tpu_sc as plsc

BVS = 1000000
MD = BVS - 1
L = 16
NC, NS = 2, 16
NW = NC * NS

CW = 262144
CWLOG = 18
NB = 8
NCH = (BVS + CW - 1) // CW


def _tc_detile_body(emt_ref, out_ref):
    x = emt_ref[...]
    out_ref[0, 0] = jnp.swapaxes(x.reshape(8, CW // 128, 128), 0, 1)


def _lookup_body(S, N, D, tk_hbm, emt_hbm, sc_hbm, out_hbm,
                 tkbuf, idxv, gbufs, obufs, scv, gsems, osems):
    CHUNK = N // NW
    wid = lax.axis_index("s") * NC + lax.axis_index("c")
    base = wid * CHUNK
    b = base // S
    s0 = base % S

    pltpu.sync_copy(sc_hbm, scv)
    pltpu.sync_copy(tk_hbm.at[pl.ds(base, CHUNK)], tkbuf.at[pl.ds(8, CHUNK)])

    @pl.when(base != 0)
    def _():
        pltpu.sync_copy(tk_hbm.at[pl.ds(base - 8, 8)], tkbuf.at[pl.ds(0, 8)])

    def hash_body(i, _):
        cur = tkbuf[pl.ds(8 + i * L, L)]
        prev = tkbuf[pl.ds(7 + i * L, L)]
        a = jnp.int32(36313) * cur
        bb = jnp.int32(27191) * prev
        r = lax.rem(lax.bitwise_xor(a, bb), jnp.int32(MD))
        r = jnp.where(r < 0, r + jnp.int32(MD), r)
        pos = base + i * L + lax.iota(jnp.int32, L)
        v = jnp.where((pos & (S - 1)) == 0, jnp.int32(MD), r)
        e = (
            lax.shift_left(lax.shift_right_logical(v, CWLOG), CWLOG + 3)
            + lax.shift_left(
                lax.bitwise_and(
                    lax.shift_right_logical(v, 7), jnp.int32(CW // 128 - 1)
                ),
                10,
            )
            + (v & 127)
        )
        for rr in range(8):
            idxv[rr, pl.ds(i * L, L)] = e + rr * 128
        return 0

    lax.fori_loop(0, CHUNK // L, hash_body, 0, unroll=2)

    scale = scv[...]
    obase0 = b * D * S + s0

    def gather_d(d, buf):
        pltpu.async_copy(
            emt_hbm.at[d // 8].at[idxv.at[d % 8]],
            gbufs.at[buf], gsems.at[buf],
        )

    for buf in range(NB):
        gather_d(buf, buf)

    def step(jo, _):
        for buf in range(NB):
            d = NB * jo + buf
            pltpu.make_async_copy(
                emt_hbm.at[d // 8].at[idxv.at[d % 8]],
                gbufs.at[buf], gsems.at[buf],
            ).wait()

            @pl.when(d >= NB)
            def _():
                pltpu.make_async_copy(
                    obufs.at[buf],
                    out_hbm.at[pl.ds(obase0 + (d - NB) * S, CHUNK)],
                    osems.at[buf],
                ).wait()

            def sbody(i, _):
                obufs[buf, pl.ds(i * L, L)] = (
                    gbufs[buf, pl.ds(i * L, L)] * scale
                )
                return 0

            lax.fori_loop(0, CHUNK // L, sbody, 0, unroll=4)

            @pl.when(d + NB < D)
            def _():
                gather_d(d + NB, buf)

            pltpu.async_copy(
                obufs.at[buf],
                out_hbm.at[pl.ds(obase0 + d * S, CHUNK)],
                osems.at[buf],
            )
        return 0

    lax.fori_loop(0, D // NB, step, 0)

    for buf in range(NB):
        pltpu.make_async_copy(
            obufs.at[buf],
            out_hbm.at[pl.ds(obase0, CHUNK)],
            osems.at[buf],
        ).wait()


def kernel(tk, em_weight, sc):
    B, S = tk.shape
    V, D = em_weight.shape
    N = B * S
    CHUNK = N // NW

    tk_flat = tk.reshape(N).astype(jnp.int32)
    emt = em_weight.T
    sc_vec = jnp.broadcast_to(sc.astype(jnp.float32), (L,))

    mesh = plsc.VectorSubcoreMesh(core_axis_name="c", subcore_axis_name="s")

    def tc_half(off):
        return pl.pallas_call(
            _tc_detile_body,
            grid=(D // 16, NCH),
            in_specs=[
                pl.BlockSpec((8, CW), lambda d8, ch: (d8 + off, ch))
            ],
            out_specs=pl.BlockSpec(
                (1, 1, CW // 128, 8, 128), lambda d8, ch: (d8, ch, 0, 0, 0)
            ),
            out_shape=jax.ShapeDtypeStruct(
                (D // 16, NCH, CW // 128, 8, 128), jnp.float32
            ),
            compiler_params=pltpu.CompilerParams(
                dimension_semantics=("arbitrary", "arbitrary"),
            ),
        )(emt)

    def sc_half(emt_lin):
        body = functools.partial(_lookup_body, S, N, D // 2)
        return pl.kernel(
            body,
            mesh=mesh,
            compiler_params=pltpu.CompilerParams(use_tc_tiling_on_sc=False),
            out_type=jax.ShapeDtypeStruct((B * (D // 2) * S,), jnp.float32),
            scratch_types=[
                pltpu.VMEM((CHUNK + 8,), jnp.int32),
                pltpu.VMEM((8, CHUNK), jnp.int32),
                pltpu.VMEM((NB, CHUNK), jnp.float32),
                pltpu.VMEM((NB, CHUNK), jnp.float32),
                pltpu.VMEM((L,), jnp.float32),
                pltpu.SemaphoreType.DMA((NB,)),
                pltpu.SemaphoreType.DMA((NB,)),
            ],
        )(tk_flat, emt_lin.reshape(D // 16, NCH * 8 * CW), sc_vec)

    halves = [sc_half(tc_half(h * (D // 16))) for h in range(2)]
    out = jnp.concatenate(
        [o.reshape(B, D // 2, S) for o in halves], axis=1
    )
    return out.transpose(0, 2, 1)
